# baseline (speedup 1.0000x reference)
import functools

import jax
import jax.numpy as jnp
from jax import lax
from jax.experimental import pallas as pl
from jax.experimental.pallas import tpu as pltpu
from jax.experimental.pallas import tpu_sc as plsc

NUM_Q = 8
CB_SIZE = 1024
D = 256
L = 16
NW = 32
C = 16
K = NUM_Q * C


def _sc_dequant(codec_wrk, lengths, emb2d, ntok, tt):
    nchunk_tot = ntok // C
    n = nchunk_tot // NW
    chunks_per_b = tt // C
    nb = ntok // tt

    @functools.partial(
        pl.kernel,
        out_type=jax.ShapeDtypeStruct((ntok, D), jnp.float32),
        mesh=plsc.VectorSubcoreMesh(core_axis_name="c", subcore_axis_name="s"),
        scratch_types=[
            pltpu.VMEM((n, K), jnp.int32),
            pltpu.VMEM((2, K, D), jnp.float32),
            pltpu.VMEM((2, C, D), jnp.float32),
            pltpu.VMEM((C, D), jnp.float32),
            pltpu.VMEM((L,), jnp.int32),
            pltpu.VMEM((L,), jnp.int32),
            pltpu.SemaphoreType.DMA,
            pltpu.SemaphoreType.DMA,
            pltpu.SemaphoreType.DMA,
            pltpu.SemaphoreType.DMA,
        ],
    )
    def body(codec_hbm, len_hbm, emb_hbm, out_hbm, codec_v, rows_v, acc_v,
             zero_v, len_v, bounce_v, gs0, gs1, os0, os1):
        gsem = (gs0, gs1)
        osem = (os0, os1)
        wid = lax.axis_index("s") * 2 + lax.axis_index("c")
        pltpu.sync_copy(len_hbm, len_v)
        lv = len_v[...]
        pltpu.sync_copy(codec_hbm.at[wid], codec_v)

        def shift_body(j, carry):
            for q in range(1, NUM_Q):
                sl = pl.ds(q * C, C)
                codec_v[j, sl] = codec_v[j, sl] + (q * CB_SIZE)
            return carry

        lax.fori_loop(0, n, shift_body, 0)

        def zfill_body(t, carry):
            for dcol in range(D // L):
                zero_v[t, pl.ds(dcol * L, L)] = jnp.zeros((L,), jnp.float32)
            return carry

        lax.fori_loop(0, C, zfill_body, 0)

        def valid_of(j):
            g = j * NW + wid
            b_idx = jnp.minimum(g // chunks_per_b, nb - 1)
            tpos = g * C - b_idx * tt
            bounce_v[...] = lv.at[jnp.full((L,), b_idx, jnp.int32)].get(
                mode="promise_in_bounds")
            len_b = bounce_v[...][0]
            return jnp.clip(len_b - tpos, 0, C)

        def fire_gather(slot, j, valid):
            @pl.when(valid > 0)
            def _():
                pltpu.async_copy(emb_hbm.at[codec_v.at[j]],
                                 rows_v.at[slot], gsem[slot])

        def drain_gather(slot, valid):
            @pl.when(valid > 0)
            def _():
                pltpu.make_async_copy(emb_hbm.at[pl.ds(0, K)],
                                      rows_v.at[slot], gsem[slot]).wait()

        def accum(slot, valid):
            @pl.when(valid >= C)
            def _full():
                def body_t(t, c2):
                    for dcol in range(D // L):
                        sl = pl.ds(dcol * L, L)
                        acc = rows_v[slot, t, sl]
                        for q in range(1, NUM_Q):
                            acc = acc + rows_v[slot, q * C + t, sl]
                        acc_v[slot, t, sl] = acc
                    return c2

                lax.fori_loop(0, C, body_t, 0)

            @pl.when(jnp.logical_and(valid > 0, valid < C))
            def _masked():
                def body_t(t, c2):
                    m = jnp.where(t < valid, 1.0, 0.0).astype(jnp.float32)
                    for dcol in range(D // L):
                        sl = pl.ds(dcol * L, L)
                        acc = rows_v[slot, t, sl]
                        for q in range(1, NUM_Q):
                            acc = acc + rows_v[slot, q * C + t, sl]
                        acc_v[slot, t, sl] = acc * m
                    return c2

                lax.fori_loop(0, C, body_t, 0)

        valid0 = valid_of(0)
        fire_gather(0, 0, valid0)

        def outer(i, vcur):
            for slot in (0, 1):
                j = 2 * i + slot
                other = 1 - slot
                vnext = valid_of(j + 1)

                @pl.when(j + 1 < n)
                def _stage_next():
                    fire_gather(other, j + 1, vnext)

                drain_gather(slot, vcur)

                @pl.when(j >= 2)
                def _drain_out():
                    pltpu.make_async_copy(
                        acc_v.at[slot],
                        out_hbm.at[pl.ds(((j - 2) * NW + wid) * C, C)],
                        osem[slot]).wait()

                accum(slot, vcur)
                off = (j * NW + wid) * C

                @pl.when(vcur > 0)
                def _store():
                    pltpu.async_copy(acc_v.at[slot],
                                     out_hbm.at[pl.ds(off, C)], osem[slot])

                @pl.when(vcur <= 0)
                def _zstore():
                    pltpu.async_copy(zero_v, out_hbm.at[pl.ds(off, C)],
                                     osem[slot])

                vcur = vnext
            return vcur

        lax.fori_loop(0, n // 2, outer, valid0)

        pltpu.make_async_copy(
            acc_v.at[0], out_hbm.at[pl.ds(((n - 2) * NW + wid) * C, C)],
            osem[0]).wait()
        pltpu.make_async_copy(
            acc_v.at[1], out_hbm.at[pl.ds(((n - 1) * NW + wid) * C, C)],
            osem[1]).wait()

    return body(codec_wrk, lengths, emb2d)


def kernel(codec, codec_lengths, embed):
    bz, tt, nq = codec.shape
    d = embed.shape[-1]
    ntok = bz * tt
    nchunk = ntok // C
    codec_wrk = (codec.reshape(nchunk, C, nq)
                 .transpose(0, 2, 1)
                 .reshape(nchunk // NW, NW, nq * C)
                 .transpose(1, 0, 2))
    out = _sc_dequant(codec_wrk, codec_lengths,
                      embed.reshape(nq * embed.shape[1], d), ntok, tt)
    return out.reshape(bz, tt, d)
